# baseline (speedup 1.0000x reference)
import functools

import jax
import jax.numpy as jnp
from jax import lax
from jax.experimental import pallas as pl
from jax.experimental.pallas import tpu as pltpu
from jax.experimental.pallas import tpu_sc as plsc

NC = 2
NS = 16
NW = NC * NS

NBUF = 8


@functools.lru_cache(maxsize=None)
def _make_lookup(batch, k, vocab, dim):
    rows_per_w = batch // NW
    assert rows_per_w % NBUF == 0

    mesh = plsc.VectorSubcoreMesh(core_axis_name="c", subcore_axis_name="s")

    @functools.partial(
        pl.kernel,
        mesh=mesh,
        compiler_params=pltpu.CompilerParams(use_tc_tiling_on_sc=False),
        out_type=jax.ShapeDtypeStruct((batch, k, dim), jnp.float32),
        scratch_types=[
            pltpu.VMEM((rows_per_w, k), jnp.int32),
            pltpu.VMEM((NBUF, k, dim), jnp.float32),
            pltpu.SemaphoreType.DMA((NBUF,)),
            pltpu.SemaphoreType.DMA((NBUF,)),
        ],
    )
    def lookup(x_hbm, w_hbm, out_hbm, idx_v, rows_v, gsem, ssem):
        wid = lax.axis_index("s") * NC + lax.axis_index("c")
        row0 = wid * rows_per_w

        pltpu.sync_copy(x_hbm.at[pl.ds(row0, rows_per_w)], idx_v)

        def gather(r, b):
            return pltpu.make_async_copy(
                w_hbm.at[idx_v.at[r]], rows_v.at[b], gsem.at[b])

        def scatter(r, b):
            return pltpu.make_async_copy(
                rows_v.at[b], out_hbm.at[row0 + r], ssem.at[b])

        for b in range(NBUF):
            gather(b, b).start()

        def group(i, _):
            i0 = i * NBUF
            for b in range(NBUF):
                r = i0 + b
                gather(r, b).wait()
                scatter(r, b).start()
                f = r + NBUF

                @pl.when(f < rows_per_w)
                def _():
                    scatter(r, b).wait()
                    gather(f, b).start()
            return 0

        lax.fori_loop(0, rows_per_w // NBUF, group, 0)

        for b in range(NBUF):
            scatter(rows_per_w - NBUF + b, b).wait()

    return lookup


def kernel(x, weight):
    batch, k = x.shape
    vocab, dim = weight.shape
    return _make_lookup(batch, k, vocab, dim)(x.astype(jnp.int32), weight)


# speedup vs baseline: 1.0758x; 1.0758x over previous
import functools

import jax
import jax.numpy as jnp
from jax import lax
from jax.experimental import pallas as pl
from jax.experimental.pallas import tpu as pltpu
from jax.experimental.pallas import tpu_sc as plsc

NC = 2
NS = 16
NW = NC * NS

NBUF = 8


@functools.lru_cache(maxsize=None)
def _make_lookup(batch, k, vocab2, dim):
    rows_per_w = batch // NW
    assert rows_per_w % NBUF == 0

    mesh = plsc.VectorSubcoreMesh(core_axis_name="c", subcore_axis_name="s")

    @functools.partial(
        pl.kernel,
        mesh=mesh,
        compiler_params=pltpu.CompilerParams(use_tc_tiling_on_sc=False),
        out_type=jax.ShapeDtypeStruct((batch * k, dim), jnp.float32),
        scratch_types=[
            pltpu.VMEM((rows_per_w, k), jnp.int32),
            pltpu.VMEM((NBUF, k, dim), jnp.float32),
            pltpu.SemaphoreType.DMA((NBUF,)),
            pltpu.SemaphoreType.DMA((NBUF,)),
        ],
    )
    def lookup(x_hbm, w_hbm, out_hbm, idx_v, rows_v, gsem, ssem):
        wid = lax.axis_index("s") * NC + lax.axis_index("c")
        row0 = wid * rows_per_w

        pltpu.sync_copy(x_hbm.at[pl.ds(row0, rows_per_w)], idx_v)

        def gather(r, b):
            return pltpu.make_async_copy(
                w_hbm.at[idx_v.at[r]], rows_v.at[b], gsem.at[b])

        def scatter(r, b):
            return pltpu.make_async_copy(
                rows_v.at[b],
                out_hbm.at[pl.ds((row0 + r) * k, k)],
                ssem.at[b])

        for b in range(NBUF):
            gather(b, b).start()

        def group(i, _):
            i0 = i * NBUF
            for b in range(NBUF):
                r = i0 + b
                gather(r, b).wait()
                scatter(r, b).start()
                f = r + NBUF

                @pl.when(f < rows_per_w)
                def _():
                    scatter(r, b).wait()
                    gather(f, b).start()
            return 0

        lax.fori_loop(0, rows_per_w // NBUF, group, 0)

        for b in range(NBUF):
            scatter(rows_per_w - NBUF + b, b).wait()

    return lookup


def kernel(x, weight):
    batch, k = x.shape
    vocab, dim = weight.shape
    wp = jnp.pad(weight, ((0, 0), (0, dim))).reshape(2 * vocab, dim)
    x2 = x.astype(jnp.int32) * 2
    out = _make_lookup(batch, k, 2 * vocab, dim)(x2, wp)
    return out.reshape(batch, k, dim)


# speedup vs baseline: 1.2978x; 1.2064x over previous
import functools

import jax
import jax.numpy as jnp
from jax import lax
from jax.experimental import pallas as pl
from jax.experimental.pallas import tpu as pltpu
from jax.experimental.pallas import tpu_sc as plsc

NC = 2
NS = 16
NW = NC * NS

NBUF = 8


@functools.lru_cache(maxsize=None)
def _make_lookup(batch, k, vocab2, dim):
    rows_per_w = batch // NW
    assert rows_per_w % NBUF == 0

    mesh = plsc.VectorSubcoreMesh(core_axis_name="c", subcore_axis_name="s")

    @functools.partial(
        pl.kernel,
        mesh=mesh,
        compiler_params=pltpu.CompilerParams(use_tc_tiling_on_sc=False),
        out_type=jax.ShapeDtypeStruct((batch, 24, 128), jnp.float32),
        scratch_types=[
            pltpu.VMEM((rows_per_w, k), jnp.int32),
            pltpu.VMEM((NBUF, k, dim), jnp.float32),
            pltpu.SemaphoreType.DMA((NBUF,)),
            pltpu.SemaphoreType.DMA((NBUF,)),
        ],
    )
    def lookup(x_hbm, w_hbm, out_hbm, idx_v, rows_v, gsem, ssem):
        wid = lax.axis_index("s") * NC + lax.axis_index("c")
        row0 = wid * rows_per_w

        pltpu.sync_copy(x_hbm.at[pl.ds(row0, rows_per_w)], idx_v)

        def gather(r, b):
            return pltpu.make_async_copy(
                w_hbm.at[idx_v.at[r]], rows_v.at[b], gsem.at[b])

        def scatter(r, b):
            return pltpu.make_async_copy(
                rows_v.at[b],
                out_hbm.at[row0 + r, pl.ds(0, k), pl.ds(0, dim)],
                ssem.at[b])

        for b in range(NBUF):
            gather(b, b).start()

        def group(i, _):
            i0 = i * NBUF
            for b in range(NBUF):
                r = i0 + b
                gather(r, b).wait()
                scatter(r, b).start()
                f = r + NBUF

                @pl.when(f < rows_per_w)
                def _():
                    scatter(r, b).wait()
                    gather(f, b).start()
            return 0

        lax.fori_loop(0, rows_per_w // NBUF, group, 0)

        for b in range(NBUF):
            scatter(rows_per_w - NBUF + b, b).wait()

    return lookup


def kernel(x, weight):
    batch, k = x.shape
    vocab, dim = weight.shape
    wp = jnp.pad(weight, ((0, 0), (0, dim))).reshape(2 * vocab, dim)
    x2 = x.astype(jnp.int32) * 2
    out = _make_lookup(batch, k, 2 * vocab, dim)(x2, wp)
    return out[:, :k, :dim]
